# baseline (speedup 1.0000x reference)
import jax
import jax.numpy as jnp
from jax import lax
from jax.experimental import pallas as pl
from jax.experimental.pallas import tpu as pltpu
from jax.experimental.pallas import tpu_sc as plsc

_N = 10000
_NPAD = 10112
_NC, _NS = 2, 16
_NW = _NC * _NS
_CH = 128
_NBUF = 4
_KWIN = 20
_KFULL = 80
_RPT = _NPAD // _NS
_D = 32

_mesh = plsc.VectorSubcoreMesh(core_axis_name="c", subcore_axis_name="s")
_sc_params = pltpu.CompilerParams(use_tc_tiling_on_sc=False)


def _deg_body(d0, d1, d2, d3, ones_hbm, zeros_hbm, out, idx_v, ones_v, acc):
    c = lax.axis_index("c")
    s = lax.axis_index("s")
    wid = c * _NS + s
    r0 = s * _RPT
    pltpu.sync_copy(ones_hbm, ones_v)
    for w in range(4):
        pltpu.sync_copy(zeros_hbm, acc.at[w, pl.ds(r0, _RPT)])
    plsc.subcore_barrier()
    for w, dref in enumerate((d0, d1, d2, d3)):
        pltpu.sync_copy(dref.at[wid], idx_v)

        def _one(j, carry, _w=w):
            pltpu.sync_copy(ones_v, acc.at[_w].at[idx_v.at[j]], add=True)
            return carry

        lax.fori_loop(0, _KWIN, _one, 0)
    plsc.subcore_barrier()
    for w in range(4):
        pltpu.sync_copy(acc.at[w, pl.ds(r0, _RPT)],
                        out.at[c, w, pl.ds(r0, _RPT)])


_deg_call = pl.kernel(
    _deg_body,
    out_type=jax.ShapeDtypeStruct((_NC, 4, _NPAD), jnp.float32),
    mesh=_mesh,
    compiler_params=_sc_params,
    scratch_types=[
        pltpu.VMEM((_KWIN, _CH), jnp.int32),
        pltpu.VMEM((_CH,), jnp.float32),
        pltpu.VMEM_SHARED((4, _NPAD), jnp.float32),
    ],
)


def _make_conv_call(Ks, inits):
    T = len(Ks)

    def body(*refs):
        srcs = [refs[3 * t] for t in range(T)]
        dsts = [refs[3 * t + 1] for t in range(T)]
        hps = [refs[3 * t + 2] for t in range(T)]
        zeros_hbm = refs[3 * T]
        outs = refs[3 * T + 1: 4 * T + 1]
        sp = 4 * T + 1
        idxs = refs[sp: sp + 2 * T]
        sp += 2 * T
        rows = refs[sp: sp + _NBUF]
        gsem = refs[sp + _NBUF: sp + 2 * _NBUF]
        ssem = refs[sp + 2 * _NBUF: sp + 3 * _NBUF]
        accs = refs[sp + 3 * _NBUF: sp + 3 * _NBUF + T]

        c = lax.axis_index("c")
        s = lax.axis_index("s")
        wid = c * _NS + s
        r0 = s * _RPT

        for t in range(T):
            if inits[t]:
                @pl.when(c == 0)
                def _(t=t):
                    pltpu.sync_copy(hps[t].at[pl.ds(r0, _RPT)],
                                    accs[t].at[pl.ds(r0, _RPT)])

                @pl.when(c != 0)
                def _(t=t):
                    pltpu.sync_copy(zeros_hbm, accs[t].at[pl.ds(r0, _RPT)])
            else:
                pltpu.sync_copy(zeros_hbm, accs[t].at[pl.ds(r0, _RPT)])
        plsc.subcore_barrier()

        for t, K in enumerate(Ks):
            isv, idv = idxs[2 * t], idxs[2 * t + 1]
            pltpu.sync_copy(srcs[t].at[wid], isv)
            pltpu.sync_copy(dsts[t].at[wid], idv)
            hp, acc = hps[t], accs[t]
            for b in range(_NBUF):
                pltpu.async_copy(hp.at[isv.at[b]], rows[b], gsem[b])

            def _group(g, carry, hp=hp, acc=acc, isv=isv, idv=idv, K=K):
                j0 = _NBUF * g
                for b in range(_NBUF):
                    pltpu.make_async_copy(hp.at[isv.at[j0 + b]],
                                          rows[b], gsem[b]).wait()
                    pltpu.async_copy(rows[b], acc.at[idv.at[j0 + b]],
                                     ssem[b], add=True)
                for b in range(_NBUF):
                    @pl.when(j0 + b + _NBUF < K)
                    def _(b=b):
                        pltpu.make_async_copy(rows[b], acc.at[idv.at[j0 + b]],
                                              ssem[b]).wait()
                        pltpu.async_copy(hp.at[isv.at[j0 + b + _NBUF]],
                                         rows[b], gsem[b])
                return carry

            lax.fori_loop(0, K // _NBUF, _group, 0)
            for b in range(_NBUF):
                pltpu.make_async_copy(rows[b], acc.at[idv.at[K - _NBUF + b]],
                                      ssem[b]).wait()

        plsc.subcore_barrier()
        for t in range(T):
            pltpu.sync_copy(accs[t].at[pl.ds(r0, _RPT)],
                            outs[t].at[c, pl.ds(r0, _RPT)])

    scratch = []
    for K in Ks:
        scratch.append(pltpu.VMEM((K, _CH), jnp.int32))
        scratch.append(pltpu.VMEM((K, _CH), jnp.int32))
    scratch += [pltpu.VMEM((_CH, _D), jnp.float32) for _ in range(_NBUF)]
    scratch += [pltpu.SemaphoreType.DMA for _ in range(2 * _NBUF)]
    scratch += [pltpu.VMEM_SHARED((_NPAD, _D), jnp.float32) for _ in range(T)]

    return pl.kernel(
        body,
        out_type=tuple(jax.ShapeDtypeStruct((_NC, _NPAD, _D), jnp.float32)
                       for _ in range(T)),
        mesh=_mesh,
        compiler_params=_sc_params,
        scratch_types=scratch,
    )


_conv_pair = _make_conv_call([_KWIN, _KWIN], [True, True])
_conv_a1 = _make_conv_call([_KWIN], [True])
_conv_a0 = _make_conv_call([_KWIN], [False])


def _epi(acc_ref, dis_col, bias, g, bvec):
    y = dis_col * (acc_ref[0] + acc_ref[1]) + bias[None, :]
    yv = y[:_N]
    m = jnp.mean(yv, axis=0)
    var = jnp.mean(yv * yv, axis=0) - m * m
    scale = lax.rsqrt(var + 1e-5) * g
    return (y - m[None, :]) * scale[None, :] + bvec[None, :]


_hp_t = jax.ShapeDtypeStruct((_NPAD, _D), jnp.float32)


def _tc0a_body(xp, l1w, l1b, cw, cfw, cbw, h_a, h_f1, h_b4):
    x1 = xp[...] @ l1w[...] + l1b[...][None, :]
    h_a[...] = x1 @ cw[...]
    h_f1[...] = x1 @ cfw[...]
    h_b4[...] = x1 @ cbw[...]


_tc0a = pl.pallas_call(_tc0a_body, out_shape=(_hp_t, _hp_t, _hp_t))


def _tc0b_body(degp, h_a, h_f1, h_b4, hp_a, hp_f1, hp_b4, dis8):
    dsum = degp[0] + degp[1]
    degf = dsum[0:1] + dsum[1:2] + dsum[2:3] + dsum[3:4] - 3.0
    dis = lax.rsqrt(jnp.concatenate(
        [dsum, degf, jnp.ones((3, _NPAD), jnp.float32)], axis=0))
    d8 = dis.T
    dis8[...] = d8
    hp_a[...] = d8[:, 4:5] * h_a[...]
    hp_f1[...] = d8[:, 0:1] * h_f1[...]
    hp_b4[...] = d8[:, 3:4] * h_b4[...]


_tc0b = pl.pallas_call(
    _tc0b_body,
    out_shape=(_hp_t, _hp_t, _hp_t,
               jax.ShapeDtypeStruct((_NPAD, 8), jnp.float32)),
)


def _make_tc_step(ecol, ncol):
    def body(acc, dis8, cb, g, bv, w, hp_n):
        d8 = dis8[...]
        xn = _epi(acc, d8[:, ecol:ecol + 1], cb[...], g[...], bv[...])
        hp_n[...] = d8[:, ncol:ncol + 1] * (xn @ w[...])

    return pl.pallas_call(body, out_shape=_hp_t)


_tc_f1 = _make_tc_step(0, 1)
_tc_f2 = _make_tc_step(1, 2)
_tc_f3 = _make_tc_step(2, 3)
_tc_b4 = _make_tc_step(3, 2)
_tc_b3 = _make_tc_step(2, 1)
_tc_b2 = _make_tc_step(1, 0)


def _make_tc_epi(ecol):
    def body(acc, dis8, cb, g, bv, xn_o):
        xn_o[...] = _epi(acc, dis8[...][:, ecol:ecol + 1], cb[...], g[...],
                         bv[...])

    return pl.pallas_call(body, out_shape=_hp_t)


_tc_f4 = _make_tc_epi(3)


def _tc_a_body(a1, a2, a3, a4, dis8, cb, g, bv, xn_o):
    asum = (a1[0] + a1[1] + a2[0] + a2[1] + a3[0] + a3[1] + a4[0] + a4[1])
    d = dis8[...][:, 4:5]
    y = d * asum + cb[...][None, :]
    yv = y[:_N]
    m = jnp.mean(yv, axis=0)
    var = jnp.mean(yv * yv, axis=0) - m * m
    scale = lax.rsqrt(var + 1e-5) * g[...]
    xn_o[...] = (y - m[None, :]) * scale[None, :] + bv[...][None, :]


_tc_a = pl.pallas_call(_tc_a_body, out_shape=_hp_t)


def _tc_fin_body(acc, dis8, cb, g, bv, xa, xf, l2w, l2b, out):
    xb = _epi(acc, dis8[...][:, 0:1], cb[...], g[...], bv[...])
    cat = jnp.concatenate([xa[...][:_N], xf[...][:_N], xb[:_N]], axis=1)
    out[...] = cat @ l2w[...] + l2b[...][None, :]


_tc_fin = pl.pallas_call(
    _tc_fin_body, out_shape=jax.ShapeDtypeStruct((_N, _D), jnp.float32))


def _pad_split(a, K):
    tot = _NW * K * _CH
    pad = jnp.full((tot - a.shape[0],), _N, jnp.int32)
    return jnp.concatenate([a.astype(jnp.int32), pad]).reshape(_NW, K, _CH)


def kernel(x, edge_index, lin1_w, lin1_b, conv_w, conv_b, convf_w, convf_b,
           convb_w, convb_b, bn_g, bn_b, bnf_g, bnf_b, bnb_g, bnb_b,
           lin2_w, lin2_b):
    ei = edge_index.astype(jnp.int32)
    src_w = [_pad_split(ei[0, w * 80000:(w + 1) * 80000], _KWIN) for w in range(4)]
    dst_w = [_pad_split(ei[1, w * 80000:(w + 1) * 80000], _KWIN) for w in range(4)]

    zeros32 = jnp.zeros((_RPT, _D), jnp.float32)
    zeros1 = jnp.zeros((_RPT,), jnp.float32)
    ones1 = jnp.ones((_CH,), jnp.float32)
    xp = jnp.concatenate([x, jnp.zeros((_NPAD - _N, x.shape[1]), x.dtype)], axis=0)

    degp = _deg_call(dst_w[0], dst_w[1], dst_w[2], dst_w[3], ones1, zeros1)
    h_a, h_f1, h_b4 = _tc0a(xp, lin1_w, lin1_b, conv_w, convf_w, convb_w)
    hp_a, hp_f1, hp_b4, dis8 = _tc0b(degp, h_a, h_f1, h_b4)

    acc_f1, acc_b4 = _conv_pair(src_w[0], dst_w[0], hp_f1,
                                src_w[3], dst_w[3], hp_b4, zeros32)
    (acc_a1,) = _conv_a1(src_w[0], dst_w[0], hp_a, zeros32)
    hp_f2 = _tc_f1(acc_f1, dis8, convf_b, bnf_g, bnf_b, convf_w)
    hp_b3 = _tc_b4(acc_b4, dis8, convb_b, bnb_g, bnb_b, convb_w)

    acc_f2, acc_b3 = _conv_pair(src_w[1], dst_w[1], hp_f2,
                                src_w[2], dst_w[2], hp_b3, zeros32)
    (acc_a2,) = _conv_a0(src_w[1], dst_w[1], hp_a, zeros32)
    hp_f3 = _tc_f2(acc_f2, dis8, convf_b, bnf_g, bnf_b, convf_w)
    hp_b2 = _tc_b3(acc_b3, dis8, convb_b, bnb_g, bnb_b, convb_w)

    acc_f3, acc_b2 = _conv_pair(src_w[2], dst_w[2], hp_f3,
                                src_w[1], dst_w[1], hp_b2, zeros32)
    (acc_a3,) = _conv_a0(src_w[2], dst_w[2], hp_a, zeros32)
    hp_f4 = _tc_f3(acc_f3, dis8, convf_b, bnf_g, bnf_b, convf_w)
    hp_b1 = _tc_b2(acc_b2, dis8, convb_b, bnb_g, bnb_b, convb_w)

    (acc_a4,) = _conv_a0(src_w[3], dst_w[3], hp_a, zeros32)
    acc_f4, acc_b1 = _conv_pair(src_w[3], dst_w[3], hp_f4,
                                src_w[0], dst_w[0], hp_b1, zeros32)
    xa = _tc_a(acc_a1, acc_a2, acc_a3, acc_a4, dis8, conv_b, bn_g, bn_b)
    xf = _tc_f4(acc_f4, dis8, convf_b, bnf_g, bnf_b)

    return _tc_fin(acc_b1, dis8, convb_b, bnb_g, bnb_b,
                   xa, xf, lin2_w, lin2_b)


# speedup vs baseline: 1.0509x; 1.0509x over previous
import jax
import jax.numpy as jnp
from jax import lax
from jax.experimental import pallas as pl
from jax.experimental.pallas import tpu as pltpu
from jax.experimental.pallas import tpu_sc as plsc

_N = 10000
_NPAD = 10112
_NC, _NS = 2, 16
_NW = _NC * _NS
_CH = 128
_KWIN = 20
_KFULL = 80
_RPT = _NPAD // _NS
_D = 32

_mesh = plsc.VectorSubcoreMesh(core_axis_name="c", subcore_axis_name="s")
_sc_params = pltpu.CompilerParams(use_tc_tiling_on_sc=False)


def _deg_body(d0, d1, d2, d3, ones_hbm, zeros_hbm, out, idx_v, ones_v, acc):
    c = lax.axis_index("c")
    s = lax.axis_index("s")
    wid = c * _NS + s
    r0 = s * _RPT
    pltpu.sync_copy(ones_hbm, ones_v)
    for w in range(4):
        pltpu.sync_copy(zeros_hbm, acc.at[w, pl.ds(r0, _RPT)])
    plsc.subcore_barrier()
    for w, dref in enumerate((d0, d1, d2, d3)):
        pltpu.sync_copy(dref.at[wid], idx_v)

        def _one(j, carry, _w=w):
            pltpu.sync_copy(ones_v, acc.at[_w].at[idx_v.at[j]], add=True)
            return carry

        lax.fori_loop(0, _KWIN, _one, 0)
    plsc.subcore_barrier()
    for w in range(4):
        pltpu.sync_copy(acc.at[w, pl.ds(r0, _RPT)],
                        out.at[c, w, pl.ds(r0, _RPT)])


_deg_call = pl.kernel(
    _deg_body,
    out_type=jax.ShapeDtypeStruct((_NC, 4, _NPAD), jnp.float32),
    mesh=_mesh,
    compiler_params=_sc_params,
    scratch_types=[
        pltpu.VMEM((_KWIN, _CH), jnp.int32),
        pltpu.VMEM((_CH,), jnp.float32),
        pltpu.VMEM_SHARED((4, _NPAD), jnp.float32),
    ],
)


def _make_conv_call(Ks):
    T = len(Ks)

    def body(*refs):
        srcs = [refs[3 * t] for t in range(T)]
        dsts = [refs[3 * t + 1] for t in range(T)]
        hps = [refs[3 * t + 2] for t in range(T)]
        zeros_hbm = refs[3 * T]
        outs = refs[3 * T + 1: 4 * T + 1]
        sp = 4 * T + 1
        idxs = refs[sp: sp + 2 * T]
        sp += 2 * T
        rows0, rows1 = refs[sp], refs[sp + 1]
        sem0, sem1 = refs[sp + 2], refs[sp + 3]
        accs = refs[sp + 4: sp + 4 + T]

        c = lax.axis_index("c")
        s = lax.axis_index("s")
        wid = c * _NS + s
        r0 = s * _RPT

        for t in range(T):
            @pl.when(c == 0)
            def _(t=t):
                pltpu.sync_copy(hps[t].at[pl.ds(r0, _RPT)],
                                accs[t].at[pl.ds(r0, _RPT)])

            @pl.when(c != 0)
            def _(t=t):
                pltpu.sync_copy(zeros_hbm, accs[t].at[pl.ds(r0, _RPT)])
        plsc.subcore_barrier()

        for t, K in enumerate(Ks):
            isv, idv = idxs[2 * t], idxs[2 * t + 1]
            pltpu.sync_copy(srcs[t].at[wid], isv)
            pltpu.sync_copy(dsts[t].at[wid], idv)
            hp, acc = hps[t], accs[t]
            pltpu.async_copy(hp.at[isv.at[0]], rows0, sem0)
            pltpu.async_copy(hp.at[isv.at[1]], rows1, sem1)

            def _pair(i, carry, hp=hp, acc=acc, isv=isv, idv=idv, K=K):
                j0 = 2 * i
                pltpu.make_async_copy(hp.at[isv.at[j0]], rows0, sem0).wait()
                pltpu.sync_copy(rows0, acc.at[idv.at[j0]], add=True)

                @pl.when(j0 + 2 < K)
                def _():
                    pltpu.async_copy(hp.at[isv.at[j0 + 2]], rows0, sem0)

                pltpu.make_async_copy(hp.at[isv.at[j0 + 1]], rows1, sem1).wait()
                pltpu.sync_copy(rows1, acc.at[idv.at[j0 + 1]], add=True)

                @pl.when(j0 + 3 < K)
                def _():
                    pltpu.async_copy(hp.at[isv.at[j0 + 3]], rows1, sem1)

                return carry

            lax.fori_loop(0, K // 2, _pair, 0)

        plsc.subcore_barrier()
        for t in range(T):
            pltpu.sync_copy(accs[t].at[pl.ds(r0, _RPT)],
                            outs[t].at[c, pl.ds(r0, _RPT)])

    scratch = []
    for K in Ks:
        scratch.append(pltpu.VMEM((K, _CH), jnp.int32))
        scratch.append(pltpu.VMEM((K, _CH), jnp.int32))
    scratch += [
        pltpu.VMEM((_CH, _D), jnp.float32),
        pltpu.VMEM((_CH, _D), jnp.float32),
        pltpu.SemaphoreType.DMA,
        pltpu.SemaphoreType.DMA,
    ]
    scratch += [pltpu.VMEM_SHARED((_NPAD, _D), jnp.float32) for _ in range(T)]

    return pl.kernel(
        body,
        out_type=tuple(jax.ShapeDtypeStruct((_NC, _NPAD, _D), jnp.float32)
                       for _ in range(T)),
        mesh=_mesh,
        compiler_params=_sc_params,
        scratch_types=scratch,
    )


_conv3 = _make_conv_call([_KFULL, _KWIN, _KWIN])
_conv2 = _make_conv_call([_KWIN, _KWIN])


def _epi(acc_ref, dis_col, bias, g, bvec):
    y = dis_col * (acc_ref[0] + acc_ref[1]) + bias[None, :]
    yv = y[:_N]
    m = jnp.mean(yv, axis=0)
    var = jnp.mean(yv * yv, axis=0) - m * m
    scale = lax.rsqrt(var + 1e-5) * g
    return (y - m[None, :]) * scale[None, :] + bvec[None, :]


_hp_t = jax.ShapeDtypeStruct((_NPAD, _D), jnp.float32)


def _tc0a_body(x, l1w, l1b, cw, cfw, cbw, h_a, h_f1, h_b4):
    x1 = x[...] @ l1w[...] + l1b[...][None, :]
    pad = jnp.zeros((_NPAD - _N, _D), jnp.float32)
    h_a[...] = jnp.concatenate([x1 @ cw[...], pad], axis=0)
    h_f1[...] = jnp.concatenate([x1 @ cfw[...], pad], axis=0)
    h_b4[...] = jnp.concatenate([x1 @ cbw[...], pad], axis=0)


_tc0a = pl.pallas_call(_tc0a_body, out_shape=(_hp_t, _hp_t, _hp_t))


def _tc0b_body(degp, h_a, h_f1, h_b4, hp_a, hp_f1, hp_b4, dis8):
    dsum = degp[0] + degp[1]
    degf = dsum[0:1] + dsum[1:2] + dsum[2:3] + dsum[3:4] - 3.0
    dis = lax.rsqrt(jnp.concatenate(
        [dsum, degf, jnp.ones((3, _NPAD), jnp.float32)], axis=0))
    d8 = dis.T
    dis8[...] = d8
    hp_a[...] = d8[:, 4:5] * h_a[...]
    hp_f1[...] = d8[:, 0:1] * h_f1[...]
    hp_b4[...] = d8[:, 3:4] * h_b4[...]


_tc0b = pl.pallas_call(
    _tc0b_body,
    out_shape=(_hp_t, _hp_t, _hp_t,
               jax.ShapeDtypeStruct((_NPAD, 8), jnp.float32)),
)


def _tc1_body(acc_a, acc_f1, acc_b4, dis8,
              conv_b, bn_g, bn_b, convf_b, bnf_g, bnf_b,
              convb_b, bnb_g, bnb_b, cfw, cbw,
              xa_o, hp_f2, hp_b3):
    d8 = dis8[...]
    xa_o[...] = _epi(acc_a, d8[:, 4:5], conv_b[...], bn_g[...], bn_b[...])
    xf = _epi(acc_f1, d8[:, 0:1], convf_b[...], bnf_g[...], bnf_b[...])
    hp_f2[...] = d8[:, 1:2] * (xf @ cfw[...])
    xb = _epi(acc_b4, d8[:, 3:4], convb_b[...], bnb_g[...], bnb_b[...])
    hp_b3[...] = d8[:, 2:3] * (xb @ cbw[...])


_tc1 = pl.pallas_call(_tc1_body, out_shape=(_hp_t, _hp_t, _hp_t))


def _make_tc_mid(fcol, bcol, fcol_next, bcol_next):
    def body(acc_f, acc_b, dis8,
             convf_b, bnf_g, bnf_b, convb_b, bnb_g, bnb_b, cfw, cbw,
             hp_f_n, hp_b_n):
        d8 = dis8[...]
        xf = _epi(acc_f, d8[:, fcol:fcol + 1],
                  convf_b[...], bnf_g[...], bnf_b[...])
        hp_f_n[...] = d8[:, fcol_next:fcol_next + 1] * (xf @ cfw[...])
        xb = _epi(acc_b, d8[:, bcol:bcol + 1],
                  convb_b[...], bnb_g[...], bnb_b[...])
        hp_b_n[...] = d8[:, bcol_next:bcol_next + 1] * (xb @ cbw[...])

    return pl.pallas_call(body, out_shape=(_hp_t, _hp_t))


_tc2 = _make_tc_mid(1, 2, 2, 1)
_tc3 = _make_tc_mid(2, 1, 3, 0)


def _tc4_body(acc_f4, acc_b1, dis8,
              convf_b, bnf_g, bnf_b, convb_b, bnb_g, bnb_b,
              xa, l2w, l2b, out):
    d8 = dis8[...]
    xf = _epi(acc_f4, d8[:, 3:4], convf_b[...], bnf_g[...], bnf_b[...])
    xb = _epi(acc_b1, d8[:, 0:1], convb_b[...], bnb_g[...], bnb_b[...])
    cat = jnp.concatenate([xa[...][:_N], xf[:_N], xb[:_N]], axis=1)
    out[...] = cat @ l2w[...] + l2b[...][None, :]


_tc4 = pl.pallas_call(
    _tc4_body, out_shape=jax.ShapeDtypeStruct((_N, _D), jnp.float32))


def _pad_split(a, K):
    tot = _NW * K * _CH
    pad = jnp.full((tot - a.shape[0],), _N, jnp.int32)
    return jnp.concatenate([a.astype(jnp.int32), pad]).reshape(_NW, K, _CH)


def kernel(x, edge_index, lin1_w, lin1_b, conv_w, conv_b, convf_w, convf_b,
           convb_w, convb_b, bn_g, bn_b, bnf_g, bnf_b, bnb_g, bnb_b,
           lin2_w, lin2_b):
    ei = edge_index.astype(jnp.int32)
    src_w = [_pad_split(ei[0, w * 80000:(w + 1) * 80000], _KWIN) for w in range(4)]
    dst_w = [_pad_split(ei[1, w * 80000:(w + 1) * 80000], _KWIN) for w in range(4)]
    src_f = _pad_split(ei[0], _KFULL)
    dst_f = _pad_split(ei[1], _KFULL)

    zeros32 = jnp.zeros((_RPT, _D), jnp.float32)
    zeros1 = jnp.zeros((_RPT,), jnp.float32)
    ones1 = jnp.ones((_CH,), jnp.float32)

    degp = _deg_call(dst_w[0], dst_w[1], dst_w[2], dst_w[3], ones1, zeros1)
    h_a, h_f1, h_b4 = _tc0a(x, lin1_w, lin1_b, conv_w, convf_w, convb_w)
    hp_a, hp_f1, hp_b4, dis8 = _tc0b(degp, h_a, h_f1, h_b4)

    acc_a, acc_f1, acc_b4 = _conv3(src_f, dst_f, hp_a,
                                   src_w[0], dst_w[0], hp_f1,
                                   src_w[3], dst_w[3], hp_b4, zeros32)

    xa, hp_f2, hp_b3 = _tc1(acc_a, acc_f1, acc_b4, dis8,
                            conv_b, bn_g, bn_b, convf_b, bnf_g, bnf_b,
                            convb_b, bnb_g, bnb_b, convf_w, convb_w)

    acc_f2, acc_b3 = _conv2(src_w[1], dst_w[1], hp_f2,
                            src_w[2], dst_w[2], hp_b3, zeros32)

    hp_f3, hp_b2 = _tc2(acc_f2, acc_b3, dis8,
                        convf_b, bnf_g, bnf_b, convb_b, bnb_g, bnb_b,
                        convf_w, convb_w)

    acc_f3, acc_b2 = _conv2(src_w[2], dst_w[2], hp_f3,
                            src_w[1], dst_w[1], hp_b2, zeros32)

    hp_f4, hp_b1 = _tc3(acc_f3, acc_b2, dis8,
                        convf_b, bnf_g, bnf_b, convb_b, bnb_g, bnb_b,
                        convf_w, convb_w)

    acc_f4, acc_b1 = _conv2(src_w[3], dst_w[3], hp_f4,
                            src_w[0], dst_w[0], hp_b1, zeros32)

    return _tc4(acc_f4, acc_b1, dis8,
                convf_b, bnf_g, bnf_b, convb_b, bnb_g, bnb_b,
                xa, lin2_w, lin2_b)


# speedup vs baseline: 1.1003x; 1.0470x over previous
import jax
import jax.numpy as jnp
from jax import lax
from jax.experimental import pallas as pl
from jax.experimental.pallas import tpu as pltpu
from jax.experimental.pallas import tpu_sc as plsc

_N = 10000
_NPAD = 10112
_NC, _NS = 2, 16
_NW = _NC * _NS
_CH = 128
_KWIN = 20
_KFULL = 80
_RPT = _NPAD // _NS
_D = 32

_mesh = plsc.VectorSubcoreMesh(core_axis_name="c", subcore_axis_name="s")
_sc_params = pltpu.CompilerParams(use_tc_tiling_on_sc=False)


def _deg_body(d0, d1, d2, d3, ones_hbm, zeros_hbm, out, idx_v, ones_v, acc):
    c = lax.axis_index("c")
    s = lax.axis_index("s")
    wid = c * _NS + s
    r0 = s * _RPT
    pltpu.sync_copy(ones_hbm, ones_v)
    for w in range(4):
        pltpu.sync_copy(zeros_hbm, acc.at[w, pl.ds(r0, _RPT)])
    plsc.subcore_barrier()
    for w, dref in enumerate((d0, d1, d2, d3)):
        pltpu.sync_copy(dref.at[wid], idx_v)

        def _one(j, carry, _w=w):
            pltpu.sync_copy(ones_v, acc.at[_w].at[idx_v.at[j]], add=True)
            return carry

        lax.fori_loop(0, _KWIN, _one, 0)
    plsc.subcore_barrier()
    for w in range(4):
        pltpu.sync_copy(acc.at[w, pl.ds(r0, _RPT)],
                        out.at[c, w, pl.ds(r0, _RPT)])


_deg_call = pl.kernel(
    _deg_body,
    out_type=jax.ShapeDtypeStruct((_NC, 4, _NPAD), jnp.float32),
    mesh=_mesh,
    compiler_params=_sc_params,
    scratch_types=[
        pltpu.VMEM((_KWIN, _CH), jnp.int32),
        pltpu.VMEM((_CH,), jnp.float32),
        pltpu.VMEM_SHARED((4, _NPAD), jnp.float32),
    ],
)


def _make_conv_call(Ks):
    T = len(Ks)

    def body(*refs):
        srcs = [refs[3 * t] for t in range(T)]
        dsts = [refs[3 * t + 1] for t in range(T)]
        hps = [refs[3 * t + 2] for t in range(T)]
        zeros_hbm = refs[3 * T]
        outs = refs[3 * T + 1: 4 * T + 1]
        sp = 4 * T + 1
        idxs = refs[sp: sp + 2 * T]
        sp += 2 * T
        rows0, rows1 = refs[sp], refs[sp + 1]
        sem0, sem1 = refs[sp + 2], refs[sp + 3]
        accs = refs[sp + 4: sp + 4 + T]

        c = lax.axis_index("c")
        s = lax.axis_index("s")
        wid = c * _NS + s
        r0 = s * _RPT

        for t in range(T):
            @pl.when(c == 0)
            def _(t=t):
                pltpu.sync_copy(hps[t].at[pl.ds(r0, _RPT)],
                                accs[t].at[pl.ds(r0, _RPT)])

            @pl.when(c != 0)
            def _(t=t):
                pltpu.sync_copy(zeros_hbm, accs[t].at[pl.ds(r0, _RPT)])
        plsc.subcore_barrier()

        for t, K in enumerate(Ks):
            isv, idv = idxs[2 * t], idxs[2 * t + 1]
            pltpu.sync_copy(srcs[t].at[wid], isv)
            pltpu.sync_copy(dsts[t].at[wid], idv)
            hp, acc = hps[t], accs[t]
            pltpu.async_copy(hp.at[isv.at[0]], rows0, sem0)
            pltpu.async_copy(hp.at[isv.at[1]], rows1, sem1)

            def _pair(i, carry, hp=hp, acc=acc, isv=isv, idv=idv, K=K):
                j0 = 2 * i
                pltpu.make_async_copy(hp.at[isv.at[j0]], rows0, sem0).wait()
                pltpu.sync_copy(rows0, acc.at[idv.at[j0]], add=True)

                @pl.when(j0 + 2 < K)
                def _():
                    pltpu.async_copy(hp.at[isv.at[j0 + 2]], rows0, sem0)

                pltpu.make_async_copy(hp.at[isv.at[j0 + 1]], rows1, sem1).wait()
                pltpu.sync_copy(rows1, acc.at[idv.at[j0 + 1]], add=True)

                @pl.when(j0 + 3 < K)
                def _():
                    pltpu.async_copy(hp.at[isv.at[j0 + 3]], rows1, sem1)

                return carry

            lax.fori_loop(0, K // 2, _pair, 0)

        plsc.subcore_barrier()
        for t in range(T):
            pltpu.sync_copy(accs[t].at[pl.ds(r0, _RPT)],
                            outs[t].at[c, pl.ds(r0, _RPT)])

    scratch = []
    for K in Ks:
        scratch.append(pltpu.VMEM((K, _CH), jnp.int32))
        scratch.append(pltpu.VMEM((K, _CH), jnp.int32))
    scratch += [
        pltpu.VMEM((_CH, _D), jnp.float32),
        pltpu.VMEM((_CH, _D), jnp.float32),
        pltpu.SemaphoreType.DMA,
        pltpu.SemaphoreType.DMA,
    ]
    scratch += [pltpu.VMEM_SHARED((_NPAD, _D), jnp.float32) for _ in range(T)]

    return pl.kernel(
        body,
        out_type=tuple(jax.ShapeDtypeStruct((_NC, _NPAD, _D), jnp.float32)
                       for _ in range(T)),
        mesh=_mesh,
        compiler_params=_sc_params,
        scratch_types=scratch,
    )


_conv3 = _make_conv_call([_KFULL, _KWIN, _KWIN])
_conv2 = _make_conv_call([_KWIN, _KWIN])


def _epi(acc_ref, dis_col, bias, g, bvec):
    y = dis_col * (acc_ref[0] + acc_ref[1]) + bias[None, :]
    yv = y[:_N]
    m = jnp.mean(yv, axis=0)
    var = jnp.mean(yv * yv, axis=0) - m * m
    scale = lax.rsqrt(var + 1e-5) * g
    return (y - m[None, :]) * scale[None, :] + bvec[None, :]


_hp_t = jax.ShapeDtypeStruct((_NPAD, _D), jnp.float32)


def _tc0_body(x, l1w, l1b, cw, cfw, cbw, degp, hp_a, hp_f1, hp_b4, dis8):
    dsum = degp[0] + degp[1]
    degf = dsum[0:1] + dsum[1:2] + dsum[2:3] + dsum[3:4] - 3.0
    dis = lax.rsqrt(jnp.concatenate(
        [dsum, degf, jnp.ones((3, _NPAD), jnp.float32)], axis=0))
    d8 = dis.T
    dis8[...] = d8
    x1 = x[...] @ l1w[...] + l1b[...][None, :]
    pad = jnp.zeros((_NPAD - _N, _D), jnp.float32)
    hp_a[...] = d8[:, 4:5] * jnp.concatenate([x1 @ cw[...], pad], axis=0)
    hp_f1[...] = d8[:, 0:1] * jnp.concatenate([x1 @ cfw[...], pad], axis=0)
    hp_b4[...] = d8[:, 3:4] * jnp.concatenate([x1 @ cbw[...], pad], axis=0)


_tc0 = pl.pallas_call(
    _tc0_body,
    out_shape=(_hp_t, _hp_t, _hp_t,
               jax.ShapeDtypeStruct((_NPAD, 8), jnp.float32)),
)


def _tc1_body(acc_a, acc_f1, acc_b4, dis8,
              conv_b, bn_g, bn_b, convf_b, bnf_g, bnf_b,
              convb_b, bnb_g, bnb_b, cfw, cbw,
              xa_o, hp_f2, hp_b3):
    d8 = dis8[...]
    xa_o[...] = _epi(acc_a, d8[:, 4:5], conv_b[...], bn_g[...], bn_b[...])
    xf = _epi(acc_f1, d8[:, 0:1], convf_b[...], bnf_g[...], bnf_b[...])
    hp_f2[...] = d8[:, 1:2] * (xf @ cfw[...])
    xb = _epi(acc_b4, d8[:, 3:4], convb_b[...], bnb_g[...], bnb_b[...])
    hp_b3[...] = d8[:, 2:3] * (xb @ cbw[...])


_tc1 = pl.pallas_call(_tc1_body, out_shape=(_hp_t, _hp_t, _hp_t))


def _make_tc_mid(fcol, bcol, fcol_next, bcol_next):
    def body(acc_f, acc_b, dis8,
             convf_b, bnf_g, bnf_b, convb_b, bnb_g, bnb_b, cfw, cbw,
             hp_f_n, hp_b_n):
        d8 = dis8[...]
        xf = _epi(acc_f, d8[:, fcol:fcol + 1],
                  convf_b[...], bnf_g[...], bnf_b[...])
        hp_f_n[...] = d8[:, fcol_next:fcol_next + 1] * (xf @ cfw[...])
        xb = _epi(acc_b, d8[:, bcol:bcol + 1],
                  convb_b[...], bnb_g[...], bnb_b[...])
        hp_b_n[...] = d8[:, bcol_next:bcol_next + 1] * (xb @ cbw[...])

    return pl.pallas_call(body, out_shape=(_hp_t, _hp_t))


_tc2 = _make_tc_mid(1, 2, 2, 1)
_tc3 = _make_tc_mid(2, 1, 3, 0)


def _tc4_body(acc_f4, acc_b1, dis8,
              convf_b, bnf_g, bnf_b, convb_b, bnb_g, bnb_b,
              xa, l2w, l2b, out):
    d8 = dis8[...]
    xf = _epi(acc_f4, d8[:, 3:4], convf_b[...], bnf_g[...], bnf_b[...])
    xb = _epi(acc_b1, d8[:, 0:1], convb_b[...], bnb_g[...], bnb_b[...])
    cat = jnp.concatenate([xa[...][:_N], xf[:_N], xb[:_N]], axis=1)
    out[...] = cat @ l2w[...] + l2b[...][None, :]


_tc4 = pl.pallas_call(
    _tc4_body, out_shape=jax.ShapeDtypeStruct((_N, _D), jnp.float32))


def _pad_split(a, K):
    tot = _NW * K * _CH
    pad = jnp.full((tot - a.shape[0],), _N, jnp.int32)
    return jnp.concatenate([a.astype(jnp.int32), pad]).reshape(_NW, K, _CH)


def kernel(x, edge_index, lin1_w, lin1_b, conv_w, conv_b, convf_w, convf_b,
           convb_w, convb_b, bn_g, bn_b, bnf_g, bnf_b, bnb_g, bnb_b,
           lin2_w, lin2_b):
    ei = edge_index.astype(jnp.int32)
    src_w = [_pad_split(ei[0, w * 80000:(w + 1) * 80000], _KWIN) for w in range(4)]
    dst_w = [_pad_split(ei[1, w * 80000:(w + 1) * 80000], _KWIN) for w in range(4)]
    src_f = _pad_split(ei[0], _KFULL)
    dst_f = _pad_split(ei[1], _KFULL)

    zeros32 = jnp.zeros((_RPT, _D), jnp.float32)
    zeros1 = jnp.zeros((_RPT,), jnp.float32)
    ones1 = jnp.ones((_CH,), jnp.float32)

    degp = _deg_call(dst_w[0], dst_w[1], dst_w[2], dst_w[3], ones1, zeros1)
    hp_a, hp_f1, hp_b4, dis8 = _tc0(x, lin1_w, lin1_b, conv_w, convf_w,
                                    convb_w, degp)

    acc_a, acc_f1, acc_b4 = _conv3(src_f, dst_f, hp_a,
                                   src_w[0], dst_w[0], hp_f1,
                                   src_w[3], dst_w[3], hp_b4, zeros32)

    xa, hp_f2, hp_b3 = _tc1(acc_a, acc_f1, acc_b4, dis8,
                            conv_b, bn_g, bn_b, convf_b, bnf_g, bnf_b,
                            convb_b, bnb_g, bnb_b, convf_w, convb_w)

    acc_f2, acc_b3 = _conv2(src_w[1], dst_w[1], hp_f2,
                            src_w[2], dst_w[2], hp_b3, zeros32)

    hp_f3, hp_b2 = _tc2(acc_f2, acc_b3, dis8,
                        convf_b, bnf_g, bnf_b, convb_b, bnb_g, bnb_b,
                        convf_w, convb_w)

    acc_f3, acc_b2 = _conv2(src_w[2], dst_w[2], hp_f3,
                            src_w[1], dst_w[1], hp_b2, zeros32)

    hp_f4, hp_b1 = _tc3(acc_f3, acc_b2, dis8,
                        convf_b, bnf_g, bnf_b, convb_b, bnb_g, bnb_b,
                        convf_w, convb_w)

    acc_f4, acc_b1 = _conv2(src_w[3], dst_w[3], hp_f4,
                            src_w[0], dst_w[0], hp_b1, zeros32)

    return _tc4(acc_f4, acc_b1, dis8,
                convf_b, bnf_g, bnf_b, convb_b, bnb_g, bnb_b,
                xa, lin2_w, lin2_b)


# speedup vs baseline: 1.1449x; 1.0405x over previous
import jax
import jax.numpy as jnp
from jax import lax
from jax.experimental import pallas as pl
from jax.experimental.pallas import tpu as pltpu
from jax.experimental.pallas import tpu_sc as plsc

_N = 10000
_NPAD = 10112
_NC, _NS = 2, 16
_NW = _NC * _NS
_CH = 128
_KWIN = 20
_KFULL = 80
_RPT = _NPAD // _NS
_D = 32

_mesh = plsc.VectorSubcoreMesh(core_axis_name="c", subcore_axis_name="s")
_sc_params = pltpu.CompilerParams(use_tc_tiling_on_sc=False)


def _deg_body(d0, d1, d2, d3, ones_hbm, zeros_hbm, out, idx_v, ones_v, acc):
    c = lax.axis_index("c")
    s = lax.axis_index("s")
    wid = c * _NS + s
    r0 = s * _RPT
    pltpu.sync_copy(ones_hbm, ones_v)
    for w in range(4):
        pltpu.sync_copy(zeros_hbm, acc.at[w, pl.ds(r0, _RPT)])
    plsc.subcore_barrier()
    for w, dref in enumerate((d0, d1, d2, d3)):
        pltpu.sync_copy(dref.at[wid], idx_v)

        def _one(j, carry, _w=w):
            pltpu.sync_copy(ones_v, acc.at[_w].at[idx_v.at[j]], add=True)
            return carry

        lax.fori_loop(0, _KWIN, _one, 0)
    plsc.subcore_barrier()
    for w in range(4):
        pltpu.sync_copy(acc.at[w, pl.ds(r0, _RPT)],
                        out.at[c, w, pl.ds(r0, _RPT)])


_deg_call = pl.kernel(
    _deg_body,
    out_type=jax.ShapeDtypeStruct((_NC, 4, _NPAD), jnp.float32),
    mesh=_mesh,
    compiler_params=_sc_params,
    scratch_types=[
        pltpu.VMEM((_KWIN, _CH), jnp.int32),
        pltpu.VMEM((_CH,), jnp.float32),
        pltpu.VMEM_SHARED((4, _NPAD), jnp.float32),
    ],
)


def _make_conv_call(Ks):
    T = len(Ks)

    def body(*refs):
        srcs = [refs[3 * t] for t in range(T)]
        dsts = [refs[3 * t + 1] for t in range(T)]
        hps = [refs[3 * t + 2] for t in range(T)]
        zeros_hbm = refs[3 * T]
        out = refs[3 * T + 1]
        sp = 3 * T + 2
        idxs = refs[sp: sp + 2 * T]
        sp += 2 * T
        rows0, rows1 = refs[sp], refs[sp + 1]
        sem0, sem1 = refs[sp + 2], refs[sp + 3]
        accs = refs[sp + 4: sp + 4 + T]

        c = lax.axis_index("c")
        s = lax.axis_index("s")
        wid = c * _NS + s
        r0 = s * _RPT

        for t in range(T):
            @pl.when(c == 0)
            def _(t=t):
                pltpu.sync_copy(hps[t].at[pl.ds(r0, _RPT)],
                                accs[t].at[pl.ds(r0, _RPT)])

            @pl.when(c != 0)
            def _(t=t):
                pltpu.sync_copy(zeros_hbm, accs[t].at[pl.ds(r0, _RPT)])
        plsc.subcore_barrier()

        for t, K in enumerate(Ks):
            isv, idv = idxs[2 * t], idxs[2 * t + 1]
            pltpu.sync_copy(srcs[t].at[wid], isv)
            pltpu.sync_copy(dsts[t].at[wid], idv)
            hp, acc = hps[t], accs[t]
            pltpu.async_copy(hp.at[isv.at[0]], rows0, sem0)
            pltpu.async_copy(hp.at[isv.at[1]], rows1, sem1)

            def _pair(i, carry, hp=hp, acc=acc, isv=isv, idv=idv, K=K):
                j0 = 2 * i
                pltpu.make_async_copy(hp.at[isv.at[j0]], rows0, sem0).wait()
                pltpu.sync_copy(rows0, acc.at[idv.at[j0]], add=True)

                @pl.when(j0 + 2 < K)
                def _():
                    pltpu.async_copy(hp.at[isv.at[j0 + 2]], rows0, sem0)

                pltpu.make_async_copy(hp.at[isv.at[j0 + 1]], rows1, sem1).wait()
                pltpu.sync_copy(rows1, acc.at[idv.at[j0 + 1]], add=True)

                @pl.when(j0 + 3 < K)
                def _():
                    pltpu.async_copy(hp.at[isv.at[j0 + 3]], rows1, sem1)

                return carry

            lax.fori_loop(0, K // 2, _pair, 0)

        plsc.subcore_barrier()
        for t in range(T):
            pltpu.sync_copy(accs[t].at[pl.ds(r0, _RPT)],
                            out.at[c, pl.ds(r0, _RPT), pl.ds(t * _D, _D)])

    scratch = []
    for K in Ks:
        scratch.append(pltpu.VMEM((K, _CH), jnp.int32))
        scratch.append(pltpu.VMEM((K, _CH), jnp.int32))
    scratch += [
        pltpu.VMEM((_CH, _D), jnp.float32),
        pltpu.VMEM((_CH, _D), jnp.float32),
        pltpu.SemaphoreType.DMA,
        pltpu.SemaphoreType.DMA,
    ]
    scratch += [pltpu.VMEM_SHARED((_NPAD, _D), jnp.float32) for _ in range(T)]

    return pl.kernel(
        body,
        out_type=jax.ShapeDtypeStruct((_NC, _NPAD, _D * T), jnp.float32),
        mesh=_mesh,
        compiler_params=_sc_params,
        scratch_types=scratch,
    )


_conv3 = _make_conv_call([_KFULL, _KWIN, _KWIN])
_conv2 = _make_conv_call([_KWIN, _KWIN])


def _epi(asum, dis_col, bias, g, bvec):
    y = dis_col * asum + bias[None, :]
    yv = y[:_N]
    m = jnp.mean(yv, axis=0)
    var = jnp.mean(yv * yv, axis=0) - m * m
    scale = lax.rsqrt(var + 1e-5) * g
    return (y - m[None, :]) * scale[None, :] + bvec[None, :]


_hp_t = jax.ShapeDtypeStruct((_NPAD, _D), jnp.float32)


def _tc0_body(x, l1w, l1b, cw, cfw, cbw, degp, hp_a, hp_f1, hp_b4, dis8):
    dsum = degp[0] + degp[1]
    degf = dsum[0:1] + dsum[1:2] + dsum[2:3] + dsum[3:4] - 3.0
    dis = lax.rsqrt(jnp.concatenate(
        [dsum, degf, jnp.ones((3, _NPAD), jnp.float32)], axis=0))
    d8 = dis.T
    dis8[...] = d8
    x1 = x[...] @ l1w[...] + l1b[...][None, :]
    pad = jnp.zeros((_NPAD - _N, _D), jnp.float32)
    hp_a[...] = d8[:, 4:5] * jnp.concatenate([x1 @ cw[...], pad], axis=0)
    hp_f1[...] = d8[:, 0:1] * jnp.concatenate([x1 @ cfw[...], pad], axis=0)
    hp_b4[...] = d8[:, 3:4] * jnp.concatenate([x1 @ cbw[...], pad], axis=0)


_tc0 = pl.pallas_call(
    _tc0_body,
    out_shape=(_hp_t, _hp_t, _hp_t,
               jax.ShapeDtypeStruct((_NPAD, 8), jnp.float32)),
)


def _tc1_body(acc, dis8,
              conv_b, bn_g, bn_b, convf_b, bnf_g, bnf_b,
              convb_b, bnb_g, bnb_b, cfw, cbw,
              xa_o, hp_f2, hp_b3):
    d8 = dis8[...]
    asum = acc[0] + acc[1]
    xa_o[...] = _epi(asum[:, 0:32], d8[:, 4:5], conv_b[...], bn_g[...], bn_b[...])
    xf = _epi(asum[:, 32:64], d8[:, 0:1], convf_b[...], bnf_g[...], bnf_b[...])
    hp_f2[...] = d8[:, 1:2] * (xf @ cfw[...])
    xb = _epi(asum[:, 64:96], d8[:, 3:4], convb_b[...], bnb_g[...], bnb_b[...])
    hp_b3[...] = d8[:, 2:3] * (xb @ cbw[...])


_tc1 = pl.pallas_call(_tc1_body, out_shape=(_hp_t, _hp_t, _hp_t))


def _make_tc_mid(fcol, bcol, fcol_next, bcol_next):
    def body(acc, dis8,
             convf_b, bnf_g, bnf_b, convb_b, bnb_g, bnb_b, cfw, cbw,
             hp_f_n, hp_b_n):
        d8 = dis8[...]
        asum = acc[0] + acc[1]
        xf = _epi(asum[:, 0:32], d8[:, fcol:fcol + 1],
                  convf_b[...], bnf_g[...], bnf_b[...])
        hp_f_n[...] = d8[:, fcol_next:fcol_next + 1] * (xf @ cfw[...])
        xb = _epi(asum[:, 32:64], d8[:, bcol:bcol + 1],
                  convb_b[...], bnb_g[...], bnb_b[...])
        hp_b_n[...] = d8[:, bcol_next:bcol_next + 1] * (xb @ cbw[...])

    return pl.pallas_call(body, out_shape=(_hp_t, _hp_t))


_tc2 = _make_tc_mid(1, 2, 2, 1)
_tc3 = _make_tc_mid(2, 1, 3, 0)


def _tc4_body(acc, dis8,
              convf_b, bnf_g, bnf_b, convb_b, bnb_g, bnb_b,
              xa, l2w, l2b, out):
    d8 = dis8[...]
    asum = acc[0] + acc[1]
    xf = _epi(asum[:, 0:32], d8[:, 3:4], convf_b[...], bnf_g[...], bnf_b[...])
    xb = _epi(asum[:, 32:64], d8[:, 0:1], convb_b[...], bnb_g[...], bnb_b[...])
    cat = jnp.concatenate([xa[...][:_N], xf[:_N], xb[:_N]], axis=1)
    out[...] = cat @ l2w[...] + l2b[...][None, :]


_tc4 = pl.pallas_call(
    _tc4_body, out_shape=jax.ShapeDtypeStruct((_N, _D), jnp.float32))


def _pad_split(a, K):
    tot = _NW * K * _CH
    pad = jnp.full((tot - a.shape[0],), _N, jnp.int32)
    return jnp.concatenate([a.astype(jnp.int32), pad]).reshape(_NW, K, _CH)


def kernel(x, edge_index, lin1_w, lin1_b, conv_w, conv_b, convf_w, convf_b,
           convb_w, convb_b, bn_g, bn_b, bnf_g, bnf_b, bnb_g, bnb_b,
           lin2_w, lin2_b):
    ei = edge_index.astype(jnp.int32)
    src_w = [_pad_split(ei[0, w * 80000:(w + 1) * 80000], _KWIN) for w in range(4)]
    dst_w = [_pad_split(ei[1, w * 80000:(w + 1) * 80000], _KWIN) for w in range(4)]
    src_f = _pad_split(ei[0], _KFULL)
    dst_f = _pad_split(ei[1], _KFULL)

    zeros32 = jnp.zeros((_RPT, _D), jnp.float32)
    zeros1 = jnp.zeros((_RPT,), jnp.float32)
    ones1 = jnp.ones((_CH,), jnp.float32)

    degp = _deg_call(dst_w[0], dst_w[1], dst_w[2], dst_w[3], ones1, zeros1)
    hp_a, hp_f1, hp_b4, dis8 = _tc0(x, lin1_w, lin1_b, conv_w, convf_w,
                                    convb_w, degp)

    acc1 = _conv3(src_f, dst_f, hp_a,
                  src_w[0], dst_w[0], hp_f1,
                  src_w[3], dst_w[3], hp_b4, zeros32)

    xa, hp_f2, hp_b3 = _tc1(acc1, dis8,
                            conv_b, bn_g, bn_b, convf_b, bnf_g, bnf_b,
                            convb_b, bnb_g, bnb_b, convf_w, convb_w)

    acc2 = _conv2(src_w[1], dst_w[1], hp_f2,
                  src_w[2], dst_w[2], hp_b3, zeros32)

    hp_f3, hp_b2 = _tc2(acc2, dis8,
                        convf_b, bnf_g, bnf_b, convb_b, bnb_g, bnb_b,
                        convf_w, convb_w)

    acc3 = _conv2(src_w[2], dst_w[2], hp_f3,
                  src_w[1], dst_w[1], hp_b2, zeros32)

    hp_f4, hp_b1 = _tc3(acc3, dis8,
                        convf_b, bnf_g, bnf_b, convb_b, bnb_g, bnb_b,
                        convf_w, convb_w)

    acc4 = _conv2(src_w[3], dst_w[3], hp_f4,
                  src_w[0], dst_w[0], hp_b1, zeros32)

    return _tc4(acc4, dis8,
                convf_b, bnf_g, bnf_b, convb_b, bnb_g, bnb_b,
                xa, lin2_w, lin2_b)


# speedup vs baseline: 1.1708x; 1.0226x over previous
import jax
import jax.numpy as jnp
from jax import lax
from jax.experimental import pallas as pl
from jax.experimental.pallas import tpu as pltpu
from jax.experimental.pallas import tpu_sc as plsc

_N = 10000
_NPAD = 10112
_NC, _NS = 2, 16
_NW = _NC * _NS
_CH = 128
_KWIN = 20
_KFULL = 80
_RPT = _NPAD // _NS
_D = 32

_mesh = plsc.VectorSubcoreMesh(core_axis_name="c", subcore_axis_name="s")
_sc_params = pltpu.CompilerParams(use_tc_tiling_on_sc=False)


def _deg_body(d0, d1, d2, d3, ones_hbm, zeros_hbm, out, idx_v, ones_v, acc):
    c = lax.axis_index("c")
    s = lax.axis_index("s")
    wid = c * _NS + s
    r0 = s * _RPT
    pltpu.sync_copy(ones_hbm, ones_v)
    for w in range(4):
        pltpu.sync_copy(zeros_hbm, acc.at[w, pl.ds(r0, _RPT)])
    plsc.subcore_barrier()
    for w, dref in enumerate((d0, d1, d2, d3)):
        pltpu.sync_copy(dref.at[wid], idx_v)

        def _one(j, carry, _w=w):
            pltpu.sync_copy(ones_v, acc.at[_w].at[idx_v.at[j]], add=True)
            return carry

        lax.fori_loop(0, _KWIN, _one, 0)
    plsc.subcore_barrier()
    for w in range(4):
        pltpu.sync_copy(acc.at[w, pl.ds(r0, _RPT)],
                        out.at[c, w, pl.ds(r0, _RPT)])


_deg_call = pl.kernel(
    _deg_body,
    out_type=jax.ShapeDtypeStruct((_NC, 4, _NPAD), jnp.float32),
    mesh=_mesh,
    compiler_params=_sc_params,
    scratch_types=[
        pltpu.VMEM((_KWIN, _CH), jnp.int32),
        pltpu.VMEM((_CH,), jnp.float32),
        pltpu.VMEM_SHARED((4, _NPAD), jnp.float32),
    ],
)


def _make_conv_call(Ks):
    T = len(Ks)

    def body(*refs):
        srcs = [refs[3 * t] for t in range(T)]
        dsts = [refs[3 * t + 1] for t in range(T)]
        hps = [refs[3 * t + 2] for t in range(T)]
        zeros_hbm = refs[3 * T]
        out = refs[3 * T + 1]
        sp = 3 * T + 2
        idxs = refs[sp: sp + 2 * T]
        sp += 2 * T
        rows0, rows1 = refs[sp], refs[sp + 1]
        sem0, sem1 = refs[sp + 2], refs[sp + 3]
        accs = refs[sp + 4: sp + 4 + T]

        c = lax.axis_index("c")
        s = lax.axis_index("s")
        wid = c * _NS + s
        r0 = s * _RPT

        for t in range(T):
            @pl.when(c == 0)
            def _(t=t):
                pltpu.sync_copy(hps[t].at[pl.ds(r0, _RPT)],
                                accs[t].at[pl.ds(r0, _RPT)])

            @pl.when(c != 0)
            def _(t=t):
                pltpu.sync_copy(zeros_hbm, accs[t].at[pl.ds(r0, _RPT)])
        plsc.subcore_barrier()

        for t, K in enumerate(Ks):
            isv, idv = idxs[2 * t], idxs[2 * t + 1]
            pltpu.sync_copy(srcs[t].at[wid], isv)
            pltpu.sync_copy(dsts[t].at[wid], idv)
            hp, acc = hps[t], accs[t]
            pltpu.async_copy(hp.at[isv.at[0]], rows0, sem0)
            pltpu.async_copy(hp.at[isv.at[1]], rows1, sem1)

            def _pair(i, carry, hp=hp, acc=acc, isv=isv, idv=idv, K=K):
                j0 = 2 * i
                pltpu.make_async_copy(hp.at[isv.at[j0]], rows0, sem0).wait()
                pltpu.sync_copy(rows0, acc.at[idv.at[j0]], add=True)

                @pl.when(j0 + 2 < K)
                def _():
                    pltpu.async_copy(hp.at[isv.at[j0 + 2]], rows0, sem0)

                pltpu.make_async_copy(hp.at[isv.at[j0 + 1]], rows1, sem1).wait()
                pltpu.sync_copy(rows1, acc.at[idv.at[j0 + 1]], add=True)

                @pl.when(j0 + 3 < K)
                def _():
                    pltpu.async_copy(hp.at[isv.at[j0 + 3]], rows1, sem1)

                return carry

            lax.fori_loop(0, K // 2, _pair, 0)

        plsc.subcore_barrier()
        for t in range(T):
            pltpu.sync_copy(accs[t].at[pl.ds(r0, _RPT)],
                            out.at[c, pl.ds(r0, _RPT), pl.ds(t * _D, _D)])

    scratch = []
    for K in Ks:
        scratch.append(pltpu.VMEM((K, _CH), jnp.int32))
        scratch.append(pltpu.VMEM((K, _CH), jnp.int32))
    scratch += [
        pltpu.VMEM((_CH, _D), jnp.float32),
        pltpu.VMEM((_CH, _D), jnp.float32),
        pltpu.SemaphoreType.DMA,
        pltpu.SemaphoreType.DMA,
    ]
    scratch += [pltpu.VMEM_SHARED((_NPAD, _D), jnp.float32) for _ in range(T)]

    return pl.kernel(
        body,
        out_type=jax.ShapeDtypeStruct((_NC, _NPAD, _D * T), jnp.float32),
        mesh=_mesh,
        compiler_params=_sc_params,
        scratch_types=scratch,
    )


_conv3 = _make_conv_call([_KFULL, _KWIN, _KWIN])
_conv2 = _make_conv_call([_KWIN, _KWIN])


def _epi(asum, dis_col, bias, g, bvec):
    y = dis_col * asum + bias[None, :]
    yv = y[:_N]
    m = jnp.mean(yv, axis=0)
    var = jnp.mean(yv * yv, axis=0) - m * m
    scale = lax.rsqrt(var + 1e-5) * g
    return (y - m[None, :]) * scale[None, :] + bvec[None, :]


_hp_t = jax.ShapeDtypeStruct((_NPAD, _D), jnp.float32)


def _tc0_body(xp, l1w, l1b, cw, cfw, cbw, degp, hp_a, hp_f1, hp_b4, dis8):
    dsum = degp[0] + degp[1]
    degf = dsum[0:1] + dsum[1:2] + dsum[2:3] + dsum[3:4] - 3.0
    dis = lax.rsqrt(jnp.concatenate(
        [dsum, degf, jnp.ones((3, _NPAD), jnp.float32)], axis=0))
    d8 = dis.T
    dis8[...] = d8
    x1 = xp[...] @ l1w[...] + l1b[...][None, :]
    hp_a[...] = d8[:, 4:5] * (x1 @ cw[...])
    hp_f1[...] = d8[:, 0:1] * (x1 @ cfw[...])
    hp_b4[...] = d8[:, 3:4] * (x1 @ cbw[...])


_tc0 = pl.pallas_call(
    _tc0_body,
    out_shape=(_hp_t, _hp_t, _hp_t,
               jax.ShapeDtypeStruct((_NPAD, 8), jnp.float32)),
)


def _tc1_body(acc, dis8,
              conv_b, bn_g, bn_b, convf_b, bnf_g, bnf_b,
              convb_b, bnb_g, bnb_b, cfw, cbw,
              xa_o, hp_f2, hp_b3):
    d8 = dis8[...]
    asum = acc[0] + acc[1]
    xa_o[...] = _epi(asum[:, 0:32], d8[:, 4:5], conv_b[...], bn_g[...], bn_b[...])
    xf = _epi(asum[:, 32:64], d8[:, 0:1], convf_b[...], bnf_g[...], bnf_b[...])
    hp_f2[...] = d8[:, 1:2] * (xf @ cfw[...])
    xb = _epi(asum[:, 64:96], d8[:, 3:4], convb_b[...], bnb_g[...], bnb_b[...])
    hp_b3[...] = d8[:, 2:3] * (xb @ cbw[...])


_tc1 = pl.pallas_call(_tc1_body, out_shape=(_hp_t, _hp_t, _hp_t))


def _make_tc_mid(fcol, bcol, fcol_next, bcol_next):
    def body(acc, dis8,
             convf_b, bnf_g, bnf_b, convb_b, bnb_g, bnb_b, cfw, cbw,
             hp_f_n, hp_b_n):
        d8 = dis8[...]
        asum = acc[0] + acc[1]
        xf = _epi(asum[:, 0:32], d8[:, fcol:fcol + 1],
                  convf_b[...], bnf_g[...], bnf_b[...])
        hp_f_n[...] = d8[:, fcol_next:fcol_next + 1] * (xf @ cfw[...])
        xb = _epi(asum[:, 32:64], d8[:, bcol:bcol + 1],
                  convb_b[...], bnb_g[...], bnb_b[...])
        hp_b_n[...] = d8[:, bcol_next:bcol_next + 1] * (xb @ cbw[...])

    return pl.pallas_call(body, out_shape=(_hp_t, _hp_t))


_tc2 = _make_tc_mid(1, 2, 2, 1)
_tc3 = _make_tc_mid(2, 1, 3, 0)


def _tc4_body(acc, dis8,
              convf_b, bnf_g, bnf_b, convb_b, bnb_g, bnb_b,
              xa, l2w, l2b, out):
    d8 = dis8[...]
    asum = acc[0] + acc[1]
    xf = _epi(asum[:, 0:32], d8[:, 3:4], convf_b[...], bnf_g[...], bnf_b[...])
    xb = _epi(asum[:, 32:64], d8[:, 0:1], convb_b[...], bnb_g[...], bnb_b[...])
    cat = jnp.concatenate([xa[...][:_N], xf[:_N], xb[:_N]], axis=1)
    out[...] = cat @ l2w[...] + l2b[...][None, :]


_tc4 = pl.pallas_call(
    _tc4_body, out_shape=jax.ShapeDtypeStruct((_N, _D), jnp.float32))


def _pad_split(a, K):
    tot = _NW * K * _CH
    pad = jnp.full((tot - a.shape[0],), _N, jnp.int32)
    return jnp.concatenate([a.astype(jnp.int32), pad]).reshape(_NW, K, _CH)


def kernel(x, edge_index, lin1_w, lin1_b, conv_w, conv_b, convf_w, convf_b,
           convb_w, convb_b, bn_g, bn_b, bnf_g, bnf_b, bnb_g, bnb_b,
           lin2_w, lin2_b):
    ei = edge_index.astype(jnp.int32)
    src_w = [_pad_split(ei[0, w * 80000:(w + 1) * 80000], _KWIN) for w in range(4)]
    dst_w = [_pad_split(ei[1, w * 80000:(w + 1) * 80000], _KWIN) for w in range(4)]
    src_f = _pad_split(ei[0], _KFULL)
    dst_f = _pad_split(ei[1], _KFULL)

    zeros32 = jnp.zeros((_RPT, _D), jnp.float32)
    zeros1 = jnp.zeros((_RPT,), jnp.float32)
    ones1 = jnp.ones((_CH,), jnp.float32)

    xp = jnp.concatenate([x, jnp.zeros((_NPAD - _N, x.shape[1]), x.dtype)],
                         axis=0)
    degp = _deg_call(dst_w[0], dst_w[1], dst_w[2], dst_w[3], ones1, zeros1)
    hp_a, hp_f1, hp_b4, dis8 = _tc0(xp, lin1_w, lin1_b, conv_w, convf_w,
                                    convb_w, degp)

    acc1 = _conv3(src_f, dst_f, hp_a,
                  src_w[0], dst_w[0], hp_f1,
                  src_w[3], dst_w[3], hp_b4, zeros32)

    xa, hp_f2, hp_b3 = _tc1(acc1, dis8,
                            conv_b, bn_g, bn_b, convf_b, bnf_g, bnf_b,
                            convb_b, bnb_g, bnb_b, convf_w, convb_w)

    acc2 = _conv2(src_w[1], dst_w[1], hp_f2,
                  src_w[2], dst_w[2], hp_b3, zeros32)

    hp_f3, hp_b2 = _tc2(acc2, dis8,
                        convf_b, bnf_g, bnf_b, convb_b, bnb_g, bnb_b,
                        convf_w, convb_w)

    acc3 = _conv2(src_w[2], dst_w[2], hp_f3,
                  src_w[1], dst_w[1], hp_b2, zeros32)

    hp_f4, hp_b1 = _tc3(acc3, dis8,
                        convf_b, bnf_g, bnf_b, convb_b, bnb_g, bnb_b,
                        convf_w, convb_w)

    acc4 = _conv2(src_w[3], dst_w[3], hp_f4,
                  src_w[0], dst_w[0], hp_b1, zeros32)

    return _tc4(acc4, dis8,
                convf_b, bnf_g, bnf_b, convb_b, bnb_g, bnb_b,
                xa, lin2_w, lin2_b)
